# baseline (speedup 1.0000x reference)
import functools

import jax
import jax.numpy as jnp
from jax import lax
from jax.experimental import pallas as pl
from jax.experimental.pallas import tpu as pltpu
from jax.experimental.pallas import tpu_sc as plsc

H_BEV = 200
W_BEV = 200
LQ = H_BEV * W_BEV
D = 256
NH = 8
NP = 4
DH = D // NH
DFFN = 1024

R_BLK = 1000

SLICES = (4000, 12000, 12000, 12000)

NW = 32
CQ = 8


def _s1v_body(inf_ref, wval_ref, bval_ref, val_out):
    val_out[...] = (
        jnp.dot(inf_ref[...].astype(jnp.bfloat16), wval_ref[...],
                preferred_element_type=jnp.float32)
        + bval_ref[...]
    )


def _stage1v(inf2, wval, bval):
    blk = lambda i: (i, 0)
    zero = lambda i: (0, 0)
    return pl.pallas_call(
        _s1v_body,
        grid=(LQ // R_BLK,),
        in_specs=[
            pl.BlockSpec((R_BLK, D), blk),
            pl.BlockSpec((D, D), zero),
            pl.BlockSpec((1, D), zero),
        ],
        out_specs=pl.BlockSpec((R_BLK, D), blk),
        out_shape=jax.ShapeDtypeStruct((LQ, D), jnp.float32),
    )(inf2, wval, bval)


def _s1i_body(q_ref, qp_ref, rp_ref, wsall_ref, ball_ref, idx_out, wgt_out):
    q = q_ref[...] + qp_ref[...]
    proj = jnp.dot(q.astype(jnp.bfloat16), wsall_ref[...],
                   preferred_element_type=jnp.float32) + ball_ref[...]
    offx = proj[:, 0:32]
    offy = proj[:, 32:64]
    att = proj[:, 64:96]
    a_p = [att[:, p * 8:(p + 1) * 8] for p in range(NP)]
    m = jnp.maximum(jnp.maximum(a_p[0], a_p[1]), jnp.maximum(a_p[2], a_p[3]))
    e_p = [jnp.exp(a - m) for a in a_p]
    s = e_p[0] + e_p[1] + e_p[2] + e_p[3]
    aw = jnp.concatenate([e / s for e in e_p], axis=1)

    rpx = rp_ref[:, 0:1] * float(W_BEV) - 0.5
    rpy = rp_ref[:, 1:2] * float(H_BEV) - 0.5
    x = rpx + offx
    y = rpy + offy
    x0 = jnp.floor(x)
    y0 = jnp.floor(y)
    fx = x - x0
    fy = y - y0
    hcol = lax.broadcasted_iota(jnp.int32, (R_BLK, 32), 1) % NH

    idx_pieces = []
    wgt_pieces = []
    for dx, dy in ((0, 0), (1, 0), (0, 1), (1, 1)):
        xi = x0 + dx
        yi = y0 + dy
        wx = (1.0 - fx) if dx == 0 else fx
        wy = (1.0 - fy) if dy == 0 else fy
        valid = ((xi >= 0) & (xi <= W_BEV - 1) & (yi >= 0) & (yi <= H_BEV - 1))
        gi = (jnp.clip(yi, 0.0, float(H_BEV - 1)) * W_BEV
              + jnp.clip(xi, 0.0, float(W_BEV - 1))).astype(jnp.int32)
        idx_pieces.append(gi * NH + hcol)
        wgt_pieces.append(wx * wy * valid.astype(jnp.float32) * aw)
    idx_out[...] = jnp.concatenate(idx_pieces, axis=1)
    wgt_out[...] = jnp.concatenate(wgt_pieces, axis=1)


def _stage1i(q2, qp2, rp2, wsall, ball, lqs):
    blk = lambda i: (i, 0)
    zero = lambda i: (0, 0)
    return pl.pallas_call(
        _s1i_body,
        grid=(lqs // R_BLK,),
        in_specs=[
            pl.BlockSpec((R_BLK, D), blk),
            pl.BlockSpec((R_BLK, D), blk),
            pl.BlockSpec((R_BLK, 2), blk),
            pl.BlockSpec((D, 96), zero),
            pl.BlockSpec((1, 96), zero),
        ],
        out_specs=[
            pl.BlockSpec((R_BLK, 128), blk),
            pl.BlockSpec((R_BLK, 128), blk),
        ],
        out_shape=[
            jax.ShapeDtypeStruct((lqs, 128), jnp.int32),
            jax.ShapeDtypeStruct((lqs, 128), jnp.float32),
        ],
    )(q2, qp2, rp2, wsall, ball)


def _sc_body(nchunk, npair, tab_hbm, idx_hbm, wgt_hbm, out_hbm, idx_v, wgt_v,
             src_v, out_v, sem0, sem1):
    wid = lax.axis_index("s") * 2 + lax.axis_index("c")
    sems = (sem0, sem1)

    def fire(ci, b):
        cid = ci * NW + wid

        @pl.when(cid < nchunk)
        def _():
            q0 = pl.multiple_of(cid * CQ, CQ)
            pltpu.sync_copy(idx_hbm.at[pl.ds(q0, CQ)], idx_v.at[b])
            pltpu.sync_copy(wgt_hbm.at[pl.ds(q0, CQ)], wgt_v.at[b])
            for j in range(CQ):
                pltpu.async_copy(tab_hbm.at[idx_v.at[b, j]],
                                 src_v.at[b, pl.ds(j * 128, 128)], sems[b])

    def drain_compute_store(ci, b):
        cid = ci * NW + wid

        @pl.when(cid < nchunk)
        def _():
            q0 = pl.multiple_of(cid * CQ, CQ)
            pltpu.make_async_copy(tab_hbm.at[pl.ds(0, CQ * 128)],
                                  src_v.at[b], sems[b]).wait()

            def qloop(ql, c2):
                wch = [wgt_v[b, ql, pl.ds(c * 16, 16)] for c in range(8)]
                for h in range(NH):
                    a0 = jnp.zeros((16,), jnp.float32)
                    a1 = jnp.zeros((16,), jnp.float32)
                    for j in range(16):
                        col = j * 8 + h
                        w = wch[col // 16][col % 16]
                        row = ql * 128 + col
                        a0 = a0 + w * src_v[b, row, pl.ds(0, 16)]
                        a1 = a1 + w * src_v[b, row, pl.ds(16, 16)]
                    out_v[b, ql, pl.ds(h * DH, 16)] = a0
                    out_v[b, ql, pl.ds(h * DH + 16, 16)] = a1
                return c2

            lax.fori_loop(0, CQ, qloop, 0)
            pltpu.sync_copy(out_v.at[b], out_hbm.at[pl.ds(q0, CQ)])

    fire(0, 0)
    fire(1, 1)

    def pairloop(pi, carry):
        ci0 = pi * 2
        drain_compute_store(ci0, 0)
        fire(ci0 + 2, 0)
        drain_compute_store(ci0 + 1, 1)
        fire(ci0 + 3, 1)
        return carry

    lax.fori_loop(0, npair, pairloop, 0)


def _stage2(table, idx, wgt, lqs):
    nchunk = lqs // CQ
    cpw = (nchunk + NW - 1) // NW
    npair = (cpw + 1) // 2
    mesh = plsc.VectorSubcoreMesh(core_axis_name="c", subcore_axis_name="s")
    k = functools.partial(
        pl.kernel,
        out_type=jax.ShapeDtypeStruct((lqs, D), jnp.float32),
        mesh=mesh,
        scratch_types=[
            pltpu.VMEM((2, CQ, 128), jnp.int32),
            pltpu.VMEM((2, CQ, 128), jnp.float32),
            pltpu.VMEM((2, CQ * 128, DH), jnp.float32),
            pltpu.VMEM((2, CQ, D), jnp.float32),
            pltpu.SemaphoreType.DMA,
            pltpu.SemaphoreType.DMA,
        ],
        compiler_params=pltpu.CompilerParams(use_tc_tiling_on_sc=False),
    )(functools.partial(_sc_body, nchunk, npair))
    return k(table, idx, wgt)


def _s3_body(att_ref, qry_ref, wao_ref, bao_ref, n1w_ref, n1b_ref,
             wf1_ref, bf1_ref, wf2_ref, bf2_ref, n2w_ref, n2b_ref, out_ref):
    o = (jnp.dot(att_ref[...].astype(jnp.bfloat16), wao_ref[...],
                 preferred_element_type=jnp.float32)
         + bao_ref[...])
    x = qry_ref[...] + o
    m = jnp.mean(x, axis=1, keepdims=True)
    xc = x - m
    v = jnp.mean(xc * xc, axis=1, keepdims=True)
    src = xc * lax.rsqrt(v + 1e-5) * n1w_ref[...] + n1b_ref[...]
    ff = jnp.maximum(
        jnp.dot(src.astype(jnp.bfloat16), wf1_ref[...],
                preferred_element_type=jnp.float32)
        + bf1_ref[...], 0.0)
    f2 = (jnp.dot(ff.astype(jnp.bfloat16), wf2_ref[...],
                  preferred_element_type=jnp.float32)
          + bf2_ref[...])
    x2 = src + f2
    m2 = jnp.mean(x2, axis=1, keepdims=True)
    xc2 = x2 - m2
    v2 = jnp.mean(xc2 * xc2, axis=1, keepdims=True)
    out_ref[...] = xc2 * lax.rsqrt(v2 + 1e-5) * n2w_ref[...] + n2b_ref[...]


def _stage3(att2, q2, wao, bao, n1w, n1b, wf1, bf1, wf2, bf2, n2w, n2b, lqs):
    blk = lambda i: (i, 0)
    zero = lambda i: (0, 0)
    return pl.pallas_call(
        _s3_body,
        grid=(lqs // R_BLK,),
        in_specs=[
            pl.BlockSpec((R_BLK, D), blk),
            pl.BlockSpec((R_BLK, D), blk),
            pl.BlockSpec((D, D), zero),
            pl.BlockSpec((1, D), zero),
            pl.BlockSpec((1, D), zero),
            pl.BlockSpec((1, D), zero),
            pl.BlockSpec((D, DFFN), zero),
            pl.BlockSpec((1, DFFN), zero),
            pl.BlockSpec((DFFN, D), zero),
            pl.BlockSpec((1, D), zero),
            pl.BlockSpec((1, D), zero),
            pl.BlockSpec((1, D), zero),
        ],
        out_specs=pl.BlockSpec((R_BLK, D), blk),
        out_shape=jax.ShapeDtypeStruct((lqs, D), jnp.float32),
    )(att2, q2, wao, bao, n1w, n1b, wf1, bf1, wf2, bf2, n2w, n2b)


def kernel(query, query_pos, reference_points, input_flatten, W_samp, b_samp,
           W_attn, b_attn, W_val, b_val, W_attn_out, b_attn_out, norm1_w,
           norm1_b, W_ff1, b_ff1, W_ff2, b_ff2, norm2_w, norm2_b):
    q2 = query[0]
    qp2 = query_pos[0]
    rp2 = reference_points[0, :, 0, :]
    inf2 = input_flatten[0]

    wsx = W_samp.reshape(D, NH, NP, 2)[..., 0].transpose(0, 2, 1).reshape(D, 32)
    wsy = W_samp.reshape(D, NH, NP, 2)[..., 1].transpose(0, 2, 1).reshape(D, 32)
    watt = W_attn.reshape(D, NH, NP).transpose(0, 2, 1).reshape(D, 32)
    bsx = b_samp.reshape(NH, NP, 2)[..., 0].T.reshape(32)
    bsy = b_samp.reshape(NH, NP, 2)[..., 1].T.reshape(32)
    batt = b_attn.reshape(NH, NP).T.reshape(32)
    wsall = jnp.concatenate([wsx, wsy, watt], axis=1)
    ball = jnp.concatenate([bsx, bsy, batt]).reshape(1, 96)
    wsall = wsall.astype(jnp.bfloat16)

    value = _stage1v(inf2, W_val.astype(jnp.bfloat16), b_val.reshape(1, D))
    table = value.reshape(LQ * NH, DH)

    wao = W_attn_out.astype(jnp.bfloat16)
    bao = b_attn_out.reshape(1, D)
    n1w = norm1_w.reshape(1, D)
    n1b = norm1_b.reshape(1, D)
    wf1 = W_ff1.astype(jnp.bfloat16)
    bf1 = b_ff1.reshape(1, DFFN)
    wf2 = W_ff2.astype(jnp.bfloat16)
    bf2 = b_ff2.reshape(1, D)
    n2w = norm2_w.reshape(1, D)
    n2b = norm2_b.reshape(1, D)

    outs = []
    off = 0
    for lqs in SLICES:
        s = slice(off, off + lqs)
        off += lqs
        idx_k, wgt_k = _stage1i(q2[s], qp2[s], rp2[s], wsall, ball, lqs)
        att_k = _stage2(table, idx_k, wgt_k, lqs)
        outs.append(_stage3(att_k, q2[s], wao, bao, n1w, n1b,
                            wf1, bf1, wf2, bf2, n2w, n2b, lqs))
    return jnp.concatenate(outs, axis=0)[None]


# speedup vs baseline: 1.0413x; 1.0413x over previous
import functools

import jax
import jax.numpy as jnp
from jax import lax
from jax.experimental import pallas as pl
from jax.experimental.pallas import tpu as pltpu
from jax.experimental.pallas import tpu_sc as plsc

H_BEV = 200
W_BEV = 200
LQ = H_BEV * W_BEV
D = 256
NH = 8
NP = 4
DH = D // NH
DFFN = 1024

R_BLK = 1000

SLICES = (2000, 8000, 10000, 10000, 10000)

NW = 32
CQ = 8


def _s1v_body(inf_ref, wval_ref, bval_ref, val_out):
    val_out[...] = (
        jnp.dot(inf_ref[...].astype(jnp.bfloat16), wval_ref[...],
                preferred_element_type=jnp.float32)
        + bval_ref[...]
    )


def _stage1v(inf2, wval, bval):
    blk = lambda i: (i, 0)
    zero = lambda i: (0, 0)
    return pl.pallas_call(
        _s1v_body,
        grid=(LQ // R_BLK,),
        in_specs=[
            pl.BlockSpec((R_BLK, D), blk),
            pl.BlockSpec((D, D), zero),
            pl.BlockSpec((1, D), zero),
        ],
        out_specs=pl.BlockSpec((R_BLK, D), blk),
        out_shape=jax.ShapeDtypeStruct((LQ, D), jnp.float32),
    )(inf2, wval, bval)


def _s1i_body(q_ref, qp_ref, rp_ref, wsall_ref, ball_ref, idx_out, wgt_out):
    q = q_ref[...] + qp_ref[...]
    proj = jnp.dot(q.astype(jnp.bfloat16), wsall_ref[...],
                   preferred_element_type=jnp.float32) + ball_ref[...]
    offx = proj[:, 0:32]
    offy = proj[:, 32:64]
    att = proj[:, 64:96]
    a_p = [att[:, p * 8:(p + 1) * 8] for p in range(NP)]
    m = jnp.maximum(jnp.maximum(a_p[0], a_p[1]), jnp.maximum(a_p[2], a_p[3]))
    e_p = [jnp.exp(a - m) for a in a_p]
    s = e_p[0] + e_p[1] + e_p[2] + e_p[3]
    aw = jnp.concatenate([e / s for e in e_p], axis=1)

    rpx = rp_ref[:, 0:1] * float(W_BEV) - 0.5
    rpy = rp_ref[:, 1:2] * float(H_BEV) - 0.5
    x = rpx + offx
    y = rpy + offy
    x0 = jnp.floor(x)
    y0 = jnp.floor(y)
    fx = x - x0
    fy = y - y0
    hcol = lax.broadcasted_iota(jnp.int32, (R_BLK, 32), 1) % NH

    idx_pieces = []
    wgt_pieces = []
    for dx, dy in ((0, 0), (1, 0), (0, 1), (1, 1)):
        xi = x0 + dx
        yi = y0 + dy
        wx = (1.0 - fx) if dx == 0 else fx
        wy = (1.0 - fy) if dy == 0 else fy
        valid = ((xi >= 0) & (xi <= W_BEV - 1) & (yi >= 0) & (yi <= H_BEV - 1))
        gi = (jnp.clip(yi, 0.0, float(H_BEV - 1)) * W_BEV
              + jnp.clip(xi, 0.0, float(W_BEV - 1))).astype(jnp.int32)
        idx_pieces.append(gi * NH + hcol)
        wgt_pieces.append(wx * wy * valid.astype(jnp.float32) * aw)
    idx_out[...] = jnp.concatenate(idx_pieces, axis=1)
    wgt_out[...] = jnp.concatenate(wgt_pieces, axis=1)


def _stage1i(q2, qp2, rp2, wsall, ball, lqs):
    blk = lambda i: (i, 0)
    zero = lambda i: (0, 0)
    return pl.pallas_call(
        _s1i_body,
        grid=(lqs // R_BLK,),
        in_specs=[
            pl.BlockSpec((R_BLK, D), blk),
            pl.BlockSpec((R_BLK, D), blk),
            pl.BlockSpec((R_BLK, 2), blk),
            pl.BlockSpec((D, 96), zero),
            pl.BlockSpec((1, 96), zero),
        ],
        out_specs=[
            pl.BlockSpec((R_BLK, 128), blk),
            pl.BlockSpec((R_BLK, 128), blk),
        ],
        out_shape=[
            jax.ShapeDtypeStruct((lqs, 128), jnp.int32),
            jax.ShapeDtypeStruct((lqs, 128), jnp.float32),
        ],
    )(q2, qp2, rp2, wsall, ball)


def _sc_body(nchunk, npair, tab_hbm, idx_hbm, wgt_hbm, out_hbm, idx_v, wgt_v,
             src_v, out_v, sem0, sem1):
    wid = lax.axis_index("s") * 2 + lax.axis_index("c")
    sems = (sem0, sem1)

    def fire(ci, b):
        cid = ci * NW + wid

        @pl.when(cid < nchunk)
        def _():
            q0 = pl.multiple_of(cid * CQ, CQ)
            pltpu.sync_copy(idx_hbm.at[pl.ds(q0, CQ)], idx_v.at[b])
            pltpu.sync_copy(wgt_hbm.at[pl.ds(q0, CQ)], wgt_v.at[b])
            for j in range(CQ):
                pltpu.async_copy(tab_hbm.at[idx_v.at[b, j]],
                                 src_v.at[b, pl.ds(j * 128, 128)], sems[b])

    def drain_compute_store(ci, b):
        cid = ci * NW + wid

        @pl.when(cid < nchunk)
        def _():
            q0 = pl.multiple_of(cid * CQ, CQ)
            pltpu.make_async_copy(tab_hbm.at[pl.ds(0, CQ * 128)],
                                  src_v.at[b], sems[b]).wait()

            def qloop(ql, c2):
                wch = [wgt_v[b, ql, pl.ds(c * 16, 16)] for c in range(8)]
                for h in range(NH):
                    a0 = jnp.zeros((16,), jnp.float32)
                    a1 = jnp.zeros((16,), jnp.float32)
                    for j in range(16):
                        col = j * 8 + h
                        w = wch[col // 16][col % 16]
                        row = ql * 128 + col
                        a0 = a0 + w * src_v[b, row, pl.ds(0, 16)]
                        a1 = a1 + w * src_v[b, row, pl.ds(16, 16)]
                    out_v[b, ql, pl.ds(h * DH, 16)] = a0
                    out_v[b, ql, pl.ds(h * DH + 16, 16)] = a1
                return c2

            lax.fori_loop(0, CQ, qloop, 0)
            pltpu.sync_copy(out_v.at[b], out_hbm.at[pl.ds(q0, CQ)])

    fire(0, 0)
    fire(1, 1)

    def pairloop(pi, carry):
        ci0 = pi * 2
        drain_compute_store(ci0, 0)
        fire(ci0 + 2, 0)
        drain_compute_store(ci0 + 1, 1)
        fire(ci0 + 3, 1)
        return carry

    lax.fori_loop(0, npair, pairloop, 0)


def _stage2(table, idx, wgt, lqs):
    nchunk = lqs // CQ
    cpw = (nchunk + NW - 1) // NW
    npair = (cpw + 1) // 2
    mesh = plsc.VectorSubcoreMesh(core_axis_name="c", subcore_axis_name="s")
    k = functools.partial(
        pl.kernel,
        out_type=jax.ShapeDtypeStruct((lqs, D), jnp.float32),
        mesh=mesh,
        scratch_types=[
            pltpu.VMEM((2, CQ, 128), jnp.int32),
            pltpu.VMEM((2, CQ, 128), jnp.float32),
            pltpu.VMEM((2, CQ * 128, DH), jnp.float32),
            pltpu.VMEM((2, CQ, D), jnp.float32),
            pltpu.SemaphoreType.DMA,
            pltpu.SemaphoreType.DMA,
        ],
        compiler_params=pltpu.CompilerParams(use_tc_tiling_on_sc=False),
    )(functools.partial(_sc_body, nchunk, npair))
    return k(table, idx, wgt)


def _s3_body(att_ref, qry_ref, wao_ref, bao_ref, n1w_ref, n1b_ref,
             wf1_ref, bf1_ref, wf2_ref, bf2_ref, n2w_ref, n2b_ref, out_ref):
    o = (jnp.dot(att_ref[...].astype(jnp.bfloat16), wao_ref[...],
                 preferred_element_type=jnp.float32)
         + bao_ref[...])
    x = qry_ref[...] + o
    m = jnp.mean(x, axis=1, keepdims=True)
    xc = x - m
    v = jnp.mean(xc * xc, axis=1, keepdims=True)
    src = xc * lax.rsqrt(v + 1e-5) * n1w_ref[...] + n1b_ref[...]
    ff = jnp.maximum(
        jnp.dot(src.astype(jnp.bfloat16), wf1_ref[...],
                preferred_element_type=jnp.float32)
        + bf1_ref[...], 0.0)
    f2 = (jnp.dot(ff.astype(jnp.bfloat16), wf2_ref[...],
                  preferred_element_type=jnp.float32)
          + bf2_ref[...])
    x2 = src + f2
    m2 = jnp.mean(x2, axis=1, keepdims=True)
    xc2 = x2 - m2
    v2 = jnp.mean(xc2 * xc2, axis=1, keepdims=True)
    out_ref[...] = xc2 * lax.rsqrt(v2 + 1e-5) * n2w_ref[...] + n2b_ref[...]


def _stage3(att2, q2, wao, bao, n1w, n1b, wf1, bf1, wf2, bf2, n2w, n2b, lqs):
    blk = lambda i: (i, 0)
    zero = lambda i: (0, 0)
    return pl.pallas_call(
        _s3_body,
        grid=(lqs // R_BLK,),
        in_specs=[
            pl.BlockSpec((R_BLK, D), blk),
            pl.BlockSpec((R_BLK, D), blk),
            pl.BlockSpec((D, D), zero),
            pl.BlockSpec((1, D), zero),
            pl.BlockSpec((1, D), zero),
            pl.BlockSpec((1, D), zero),
            pl.BlockSpec((D, DFFN), zero),
            pl.BlockSpec((1, DFFN), zero),
            pl.BlockSpec((DFFN, D), zero),
            pl.BlockSpec((1, D), zero),
            pl.BlockSpec((1, D), zero),
            pl.BlockSpec((1, D), zero),
        ],
        out_specs=pl.BlockSpec((R_BLK, D), blk),
        out_shape=jax.ShapeDtypeStruct((lqs, D), jnp.float32),
    )(att2, q2, wao, bao, n1w, n1b, wf1, bf1, wf2, bf2, n2w, n2b)


def kernel(query, query_pos, reference_points, input_flatten, W_samp, b_samp,
           W_attn, b_attn, W_val, b_val, W_attn_out, b_attn_out, norm1_w,
           norm1_b, W_ff1, b_ff1, W_ff2, b_ff2, norm2_w, norm2_b):
    q2 = query[0]
    qp2 = query_pos[0]
    rp2 = reference_points[0, :, 0, :]
    inf2 = input_flatten[0]

    wsx = W_samp.reshape(D, NH, NP, 2)[..., 0].transpose(0, 2, 1).reshape(D, 32)
    wsy = W_samp.reshape(D, NH, NP, 2)[..., 1].transpose(0, 2, 1).reshape(D, 32)
    watt = W_attn.reshape(D, NH, NP).transpose(0, 2, 1).reshape(D, 32)
    bsx = b_samp.reshape(NH, NP, 2)[..., 0].T.reshape(32)
    bsy = b_samp.reshape(NH, NP, 2)[..., 1].T.reshape(32)
    batt = b_attn.reshape(NH, NP).T.reshape(32)
    wsall = jnp.concatenate([wsx, wsy, watt], axis=1)
    ball = jnp.concatenate([bsx, bsy, batt]).reshape(1, 96)
    wsall = wsall.astype(jnp.bfloat16)

    value = _stage1v(inf2, W_val.astype(jnp.bfloat16), b_val.reshape(1, D))
    table = value.reshape(LQ * NH, DH)

    wao = W_attn_out.astype(jnp.bfloat16)
    bao = b_attn_out.reshape(1, D)
    n1w = norm1_w.reshape(1, D)
    n1b = norm1_b.reshape(1, D)
    wf1 = W_ff1.astype(jnp.bfloat16)
    bf1 = b_ff1.reshape(1, DFFN)
    wf2 = W_ff2.astype(jnp.bfloat16)
    bf2 = b_ff2.reshape(1, D)
    n2w = norm2_w.reshape(1, D)
    n2b = norm2_b.reshape(1, D)

    outs = []
    off = 0
    for lqs in SLICES:
        s = slice(off, off + lqs)
        off += lqs
        idx_k, wgt_k = _stage1i(q2[s], qp2[s], rp2[s], wsall, ball, lqs)
        att_k = _stage2(table, idx_k, wgt_k, lqs)
        outs.append(_stage3(att_k, q2[s], wao, bao, n1w, n1b,
                            wf1, bf1, wf2, bf2, n2w, n2b, lqs))
    return jnp.concatenate(outs, axis=0)[None]


# speedup vs baseline: 1.0534x; 1.0116x over previous
import functools

import jax
import jax.numpy as jnp
from jax import lax
from jax.experimental import pallas as pl
from jax.experimental.pallas import tpu as pltpu
from jax.experimental.pallas import tpu_sc as plsc

H_BEV = 200
W_BEV = 200
LQ = H_BEV * W_BEV
D = 256
NH = 8
NP = 4
DH = D // NH
DFFN = 1024

R_BLK = 1000

SLICES = (1000, 4000, 8000, 9000, 9000, 9000)

NW = 32
CQ = 8


def _s1v_body(inf_ref, wval_ref, bval_ref, val_out):
    val_out[...] = (
        jnp.dot(inf_ref[...].astype(jnp.bfloat16), wval_ref[...],
                preferred_element_type=jnp.float32)
        + bval_ref[...]
    )


def _stage1v(inf2, wval, bval):
    blk = lambda i: (i, 0)
    zero = lambda i: (0, 0)
    return pl.pallas_call(
        _s1v_body,
        grid=(LQ // R_BLK,),
        in_specs=[
            pl.BlockSpec((R_BLK, D), blk),
            pl.BlockSpec((D, D), zero),
            pl.BlockSpec((1, D), zero),
        ],
        out_specs=pl.BlockSpec((R_BLK, D), blk),
        out_shape=jax.ShapeDtypeStruct((LQ, D), jnp.float32),
    )(inf2, wval, bval)


def _s1i_body(q_ref, qp_ref, rp_ref, wsall_ref, ball_ref, idx_out, wgt_out):
    q = q_ref[...] + qp_ref[...]
    proj = jnp.dot(q.astype(jnp.bfloat16), wsall_ref[...],
                   preferred_element_type=jnp.float32) + ball_ref[...]
    offx = proj[:, 0:32]
    offy = proj[:, 32:64]
    att = proj[:, 64:96]
    a_p = [att[:, p * 8:(p + 1) * 8] for p in range(NP)]
    m = jnp.maximum(jnp.maximum(a_p[0], a_p[1]), jnp.maximum(a_p[2], a_p[3]))
    e_p = [jnp.exp(a - m) for a in a_p]
    s = e_p[0] + e_p[1] + e_p[2] + e_p[3]
    aw = jnp.concatenate([e / s for e in e_p], axis=1)

    rpx = rp_ref[:, 0:1] * float(W_BEV) - 0.5
    rpy = rp_ref[:, 1:2] * float(H_BEV) - 0.5
    x = rpx + offx
    y = rpy + offy
    x0 = jnp.floor(x)
    y0 = jnp.floor(y)
    fx = x - x0
    fy = y - y0
    hcol = lax.broadcasted_iota(jnp.int32, (R_BLK, 32), 1) % NH

    idx_pieces = []
    wgt_pieces = []
    for dx, dy in ((0, 0), (1, 0), (0, 1), (1, 1)):
        xi = x0 + dx
        yi = y0 + dy
        wx = (1.0 - fx) if dx == 0 else fx
        wy = (1.0 - fy) if dy == 0 else fy
        valid = ((xi >= 0) & (xi <= W_BEV - 1) & (yi >= 0) & (yi <= H_BEV - 1))
        gi = (jnp.clip(yi, 0.0, float(H_BEV - 1)) * W_BEV
              + jnp.clip(xi, 0.0, float(W_BEV - 1))).astype(jnp.int32)
        idx_pieces.append(gi * NH + hcol)
        wgt_pieces.append(wx * wy * valid.astype(jnp.float32) * aw)
    idx_out[...] = jnp.concatenate(idx_pieces, axis=1)
    wgt_out[...] = jnp.concatenate(wgt_pieces, axis=1)


def _stage1i(q2, qp2, rp2, wsall, ball, lqs):
    blk = lambda i: (i, 0)
    zero = lambda i: (0, 0)
    return pl.pallas_call(
        _s1i_body,
        grid=(lqs // R_BLK,),
        in_specs=[
            pl.BlockSpec((R_BLK, D), blk),
            pl.BlockSpec((R_BLK, D), blk),
            pl.BlockSpec((R_BLK, 2), blk),
            pl.BlockSpec((D, 96), zero),
            pl.BlockSpec((1, 96), zero),
        ],
        out_specs=[
            pl.BlockSpec((R_BLK, 128), blk),
            pl.BlockSpec((R_BLK, 128), blk),
        ],
        out_shape=[
            jax.ShapeDtypeStruct((lqs, 128), jnp.int32),
            jax.ShapeDtypeStruct((lqs, 128), jnp.float32),
        ],
    )(q2, qp2, rp2, wsall, ball)


def _sc_body(nchunk, npair, tab_hbm, idx_hbm, wgt_hbm, out_hbm, idx_v, wgt_v,
             src_v, out_v, sem0, sem1):
    wid = lax.axis_index("s") * 2 + lax.axis_index("c")
    sems = (sem0, sem1)

    def fire(ci, b):
        cid = ci * NW + wid

        @pl.when(cid < nchunk)
        def _():
            q0 = pl.multiple_of(cid * CQ, CQ)
            pltpu.sync_copy(idx_hbm.at[pl.ds(q0, CQ)], idx_v.at[b])
            pltpu.sync_copy(wgt_hbm.at[pl.ds(q0, CQ)], wgt_v.at[b])
            for j in range(CQ):
                pltpu.async_copy(tab_hbm.at[idx_v.at[b, j]],
                                 src_v.at[b, pl.ds(j * 128, 128)], sems[b])

    def drain_compute_store(ci, b):
        cid = ci * NW + wid

        @pl.when(cid < nchunk)
        def _():
            q0 = pl.multiple_of(cid * CQ, CQ)
            pltpu.make_async_copy(tab_hbm.at[pl.ds(0, CQ * 128)],
                                  src_v.at[b], sems[b]).wait()

            def qloop(ql, c2):
                wch = [wgt_v[b, ql, pl.ds(c * 16, 16)] for c in range(8)]
                for h in range(NH):
                    a0 = jnp.zeros((16,), jnp.float32)
                    a1 = jnp.zeros((16,), jnp.float32)
                    for j in range(16):
                        col = j * 8 + h
                        w = wch[col // 16][col % 16]
                        row = ql * 128 + col
                        a0 = a0 + w * src_v[b, row, pl.ds(0, 16)]
                        a1 = a1 + w * src_v[b, row, pl.ds(16, 16)]
                    out_v[b, ql, pl.ds(h * DH, 16)] = a0
                    out_v[b, ql, pl.ds(h * DH + 16, 16)] = a1
                return c2

            lax.fori_loop(0, CQ, qloop, 0)
            pltpu.sync_copy(out_v.at[b], out_hbm.at[pl.ds(q0, CQ)])

    fire(0, 0)
    fire(1, 1)

    def pairloop(pi, carry):
        ci0 = pi * 2
        drain_compute_store(ci0, 0)
        fire(ci0 + 2, 0)
        drain_compute_store(ci0 + 1, 1)
        fire(ci0 + 3, 1)
        return carry

    lax.fori_loop(0, npair, pairloop, 0)


def _stage2(table, idx, wgt, lqs):
    nchunk = lqs // CQ
    cpw = (nchunk + NW - 1) // NW
    npair = (cpw + 1) // 2
    mesh = plsc.VectorSubcoreMesh(core_axis_name="c", subcore_axis_name="s")
    k = functools.partial(
        pl.kernel,
        out_type=jax.ShapeDtypeStruct((lqs, D), jnp.float32),
        mesh=mesh,
        scratch_types=[
            pltpu.VMEM((2, CQ, 128), jnp.int32),
            pltpu.VMEM((2, CQ, 128), jnp.float32),
            pltpu.VMEM((2, CQ * 128, DH), jnp.float32),
            pltpu.VMEM((2, CQ, D), jnp.float32),
            pltpu.SemaphoreType.DMA,
            pltpu.SemaphoreType.DMA,
        ],
        compiler_params=pltpu.CompilerParams(use_tc_tiling_on_sc=False),
    )(functools.partial(_sc_body, nchunk, npair))
    return k(table, idx, wgt)


def _s3_body(att_ref, qry_ref, wao_ref, bao_ref, n1w_ref, n1b_ref,
             wf1_ref, bf1_ref, wf2_ref, bf2_ref, n2w_ref, n2b_ref, out_ref):
    o = (jnp.dot(att_ref[...].astype(jnp.bfloat16), wao_ref[...],
                 preferred_element_type=jnp.float32)
         + bao_ref[...])
    x = qry_ref[...] + o
    m = jnp.mean(x, axis=1, keepdims=True)
    xc = x - m
    v = jnp.mean(xc * xc, axis=1, keepdims=True)
    src = xc * lax.rsqrt(v + 1e-5) * n1w_ref[...] + n1b_ref[...]
    ff = jnp.maximum(
        jnp.dot(src.astype(jnp.bfloat16), wf1_ref[...],
                preferred_element_type=jnp.float32)
        + bf1_ref[...], 0.0)
    f2 = (jnp.dot(ff.astype(jnp.bfloat16), wf2_ref[...],
                  preferred_element_type=jnp.float32)
          + bf2_ref[...])
    x2 = src + f2
    m2 = jnp.mean(x2, axis=1, keepdims=True)
    xc2 = x2 - m2
    v2 = jnp.mean(xc2 * xc2, axis=1, keepdims=True)
    out_ref[...] = xc2 * lax.rsqrt(v2 + 1e-5) * n2w_ref[...] + n2b_ref[...]


def _stage3(att2, q2, wao, bao, n1w, n1b, wf1, bf1, wf2, bf2, n2w, n2b, lqs):
    blk = lambda i: (i, 0)
    zero = lambda i: (0, 0)
    return pl.pallas_call(
        _s3_body,
        grid=(lqs // R_BLK,),
        in_specs=[
            pl.BlockSpec((R_BLK, D), blk),
            pl.BlockSpec((R_BLK, D), blk),
            pl.BlockSpec((D, D), zero),
            pl.BlockSpec((1, D), zero),
            pl.BlockSpec((1, D), zero),
            pl.BlockSpec((1, D), zero),
            pl.BlockSpec((D, DFFN), zero),
            pl.BlockSpec((1, DFFN), zero),
            pl.BlockSpec((DFFN, D), zero),
            pl.BlockSpec((1, D), zero),
            pl.BlockSpec((1, D), zero),
            pl.BlockSpec((1, D), zero),
        ],
        out_specs=pl.BlockSpec((R_BLK, D), blk),
        out_shape=jax.ShapeDtypeStruct((lqs, D), jnp.float32),
    )(att2, q2, wao, bao, n1w, n1b, wf1, bf1, wf2, bf2, n2w, n2b)


def kernel(query, query_pos, reference_points, input_flatten, W_samp, b_samp,
           W_attn, b_attn, W_val, b_val, W_attn_out, b_attn_out, norm1_w,
           norm1_b, W_ff1, b_ff1, W_ff2, b_ff2, norm2_w, norm2_b):
    q2 = query[0]
    qp2 = query_pos[0]
    rp2 = reference_points[0, :, 0, :]
    inf2 = input_flatten[0]

    wsx = W_samp.reshape(D, NH, NP, 2)[..., 0].transpose(0, 2, 1).reshape(D, 32)
    wsy = W_samp.reshape(D, NH, NP, 2)[..., 1].transpose(0, 2, 1).reshape(D, 32)
    watt = W_attn.reshape(D, NH, NP).transpose(0, 2, 1).reshape(D, 32)
    bsx = b_samp.reshape(NH, NP, 2)[..., 0].T.reshape(32)
    bsy = b_samp.reshape(NH, NP, 2)[..., 1].T.reshape(32)
    batt = b_attn.reshape(NH, NP).T.reshape(32)
    wsall = jnp.concatenate([wsx, wsy, watt], axis=1)
    ball = jnp.concatenate([bsx, bsy, batt]).reshape(1, 96)
    wsall = wsall.astype(jnp.bfloat16)

    value = _stage1v(inf2, W_val.astype(jnp.bfloat16), b_val.reshape(1, D))
    table = value.reshape(LQ * NH, DH)

    wao = W_attn_out.astype(jnp.bfloat16)
    bao = b_attn_out.reshape(1, D)
    n1w = norm1_w.reshape(1, D)
    n1b = norm1_b.reshape(1, D)
    wf1 = W_ff1.astype(jnp.bfloat16)
    bf1 = b_ff1.reshape(1, DFFN)
    wf2 = W_ff2.astype(jnp.bfloat16)
    bf2 = b_ff2.reshape(1, D)
    n2w = norm2_w.reshape(1, D)
    n2b = norm2_b.reshape(1, D)

    outs = []
    off = 0
    for lqs in SLICES:
        s = slice(off, off + lqs)
        off += lqs
        idx_k, wgt_k = _stage1i(q2[s], qp2[s], rp2[s], wsall, ball, lqs)
        att_k = _stage2(table, idx_k, wgt_k, lqs)
        outs.append(_stage3(att_k, q2[s], wao, bao, n1w, n1b,
                            wf1, bf1, wf2, bf2, n2w, n2b, lqs))
    return jnp.concatenate(outs, axis=0)[None]
